# Initial kernel scaffold; baseline (speedup 1.0000x reference)
import jax
import jax.numpy as jnp
from jax.experimental import pallas as pl


def kernel(x, lap_coefs, mf_weights, edge_index):
    raise NotImplementedError("write your pallas kernel here")


# speedup vs baseline: 9.4753x; 9.4753x over previous
import functools

import jax
import jax.numpy as jnp
from jax import lax
from jax.experimental import pallas as pl
from jax.experimental.pallas import tpu as pltpu
from jax.experimental.pallas import tpu_sc as plsc

K = 10
N_NODES = 10000
D = 128
NP = 10240
N_EDGES = 320000
NC, NS = 2, 16
W = NC * NS
B = 128
CH = -(-N_EDGES // (W * B))
EP = W * CH * B
ROWS_PER_TILE = NP // NS

_mesh = plsc.VectorSubcoreMesh(
    core_axis_name="c", subcore_axis_name="s", num_cores=NC, num_subcores=NS)


@functools.partial(
    pl.kernel,
    out_type=jax.ShapeDtypeStruct((NC, NP), jnp.float32),
    mesh=_mesh,
    scratch_types=[
        pltpu.VMEM((CH, B), jnp.int32),
        pltpu.VMEM((B,), jnp.float32),
        pltpu.VMEM_SHARED((NP,), jnp.float32),
    ],
)
def _deg_kernel(dst_hbm, ones_hbm, zeros_hbm, out_hbm, dst_v, ones_v, deg_sh):
    c = lax.axis_index("c")
    s = lax.axis_index("s")
    wid = c * NS + s
    pltpu.sync_copy(zeros_hbm.at[pl.ds(s * ROWS_PER_TILE, ROWS_PER_TILE)],
                    deg_sh.at[pl.ds(s * ROWS_PER_TILE, ROWS_PER_TILE)])
    pltpu.sync_copy(dst_hbm.at[wid], dst_v)
    pltpu.sync_copy(ones_hbm, ones_v)
    plsc.subcore_barrier()

    def body(j, carry):
        pltpu.sync_copy(ones_v, deg_sh.at[dst_v.at[j]], add=True)
        return carry

    lax.fori_loop(0, CH, body, 0)
    plsc.subcore_barrier()
    pltpu.sync_copy(deg_sh.at[pl.ds(s * ROWS_PER_TILE, ROWS_PER_TILE)],
                    out_hbm.at[c, pl.ds(s * ROWS_PER_TILE, ROWS_PER_TILE)])


@functools.partial(
    pl.kernel,
    out_type=jax.ShapeDtypeStruct((NC, NP, D), jnp.float32),
    mesh=_mesh,
    scratch_types=[
        pltpu.VMEM((CH, B), jnp.int32),
        pltpu.VMEM((CH, B), jnp.int32),
        pltpu.VMEM((B, D), jnp.float32),
        pltpu.VMEM_SHARED((NP, D), jnp.float32),
    ],
)
def _edge_kernel(g_hbm, src_hbm, dst_hbm, zeros_hbm, out_hbm,
                 src_v, dst_v, gbuf, acc_sh):
    c = lax.axis_index("c")
    s = lax.axis_index("s")
    wid = c * NS + s
    pltpu.sync_copy(zeros_hbm.at[pl.ds(s * ROWS_PER_TILE, ROWS_PER_TILE)],
                    acc_sh.at[pl.ds(s * ROWS_PER_TILE, ROWS_PER_TILE)])
    pltpu.sync_copy(src_hbm.at[wid], src_v)
    pltpu.sync_copy(dst_hbm.at[wid], dst_v)
    plsc.subcore_barrier()

    def body(j, carry):
        pltpu.sync_copy(g_hbm.at[src_v.at[j]], gbuf)
        pltpu.sync_copy(gbuf, acc_sh.at[dst_v.at[j]], add=True)
        return carry

    lax.fori_loop(0, CH, body, 0)
    plsc.subcore_barrier()
    pltpu.sync_copy(acc_sh.at[pl.ds(s * ROWS_PER_TILE, ROWS_PER_TILE)],
                    out_hbm.at[c, pl.ds(s * ROWS_PER_TILE, ROWS_PER_TILE)])


def _setup_body(deg_ref, x_ref, c0_ref, ds_ref, g_ref, acc_ref):
    d = deg_ref[0] + deg_ref[1]
    ds = jnp.where(d > 0.0, lax.rsqrt(jnp.maximum(d, 1e-30)), 0.0)
    ds_ref[...] = ds
    g_ref[...] = ds * x_ref[...]
    acc_ref[...] = c0_ref[0, 0] * x_ref[...]


def _combine_body(a, b, p_ref, ds_ref, tm2_ref, acc_ref, ci_ref,
                  t_ref, g_ref, acco_ref):
    m = p_ref[0] + p_ref[1]
    ds = ds_ref[...]
    t = (a * ds) * m + b * tm2_ref[...]
    acco_ref[...] = acc_ref[...] + ci_ref[0, 0] * t
    t_ref[...] = t
    g_ref[...] = ds * t


_BLK = 256
_GRID = NP // _BLK


def _tc_setup(deg2, x_pad, c0):
    return pl.pallas_call(
        _setup_body,
        grid=(_GRID,),
        in_specs=[
            pl.BlockSpec((NC, _BLK, 1), lambda i: (0, i, 0)),
            pl.BlockSpec((_BLK, D), lambda i: (i, 0)),
            pl.BlockSpec((1, 1), lambda i: (0, 0)),
        ],
        out_specs=[
            pl.BlockSpec((_BLK, 1), lambda i: (i, 0)),
            pl.BlockSpec((_BLK, D), lambda i: (i, 0)),
            pl.BlockSpec((_BLK, D), lambda i: (i, 0)),
        ],
        out_shape=[
            jax.ShapeDtypeStruct((NP, 1), jnp.float32),
            jax.ShapeDtypeStruct((NP, D), jnp.float32),
            jax.ShapeDtypeStruct((NP, D), jnp.float32),
        ],
    )(deg2, x_pad, c0)


def _tc_combine(a, b, partial, ds, t_m2, acc, ci):
    return pl.pallas_call(
        functools.partial(_combine_body, a, b),
        grid=(_GRID,),
        in_specs=[
            pl.BlockSpec((NC, _BLK, D), lambda i: (0, i, 0)),
            pl.BlockSpec((_BLK, 1), lambda i: (i, 0)),
            pl.BlockSpec((_BLK, D), lambda i: (i, 0)),
            pl.BlockSpec((_BLK, D), lambda i: (i, 0)),
            pl.BlockSpec((1, 1), lambda i: (0, 0)),
        ],
        out_specs=[
            pl.BlockSpec((_BLK, D), lambda i: (i, 0)),
            pl.BlockSpec((_BLK, D), lambda i: (i, 0)),
            pl.BlockSpec((_BLK, D), lambda i: (i, 0)),
        ],
        out_shape=[
            jax.ShapeDtypeStruct((NP, D), jnp.float32),
            jax.ShapeDtypeStruct((NP, D), jnp.float32),
            jax.ShapeDtypeStruct((NP, D), jnp.float32),
        ],
    )(partial, ds, t_m2, acc, ci)


def kernel(x, lap_coefs, mf_weights, edge_index):
    src = edge_index[0].astype(jnp.int32)
    dst = edge_index[1].astype(jnp.int32)
    npad = EP - N_EDGES
    pad_src = (jnp.arange(npad, dtype=jnp.int32) * 997) % N_NODES
    pad_dst = N_NODES + (jnp.arange(npad, dtype=jnp.int32) % (NP - N_NODES))
    src3 = jnp.concatenate([src, pad_src]).reshape(W, CH, B)
    dst3 = jnp.concatenate([dst, pad_dst]).reshape(W, CH, B)

    x_pad = jnp.zeros((NP, D), jnp.float32).at[:N_NODES].set(x)
    zeros2 = jnp.zeros((NP, D), jnp.float32)
    zeros1 = jnp.zeros((NP,), jnp.float32)
    ones_b = jnp.ones((B,), jnp.float32)

    lc = jnp.cumprod(jnp.tanh(lap_coefs))
    mw = mf_weights[0, :, 0]
    coefs = mw * jnp.concatenate([jnp.ones((1,), jnp.float32), lc[:-1]])
    c_arr = coefs.reshape(K + 1, 1, 1)

    deg2 = _deg_kernel(dst3, ones_b, zeros1)
    deg2 = deg2.reshape(NC, NP, 1)
    ds, g, acc = _tc_setup(deg2, x_pad, c_arr[0])

    t_im1, t_im2 = x_pad, x_pad
    for i in range(1, K + 1):
        partial = _edge_kernel(g, src3, dst3, zeros2)
        a, b = (1.0, 0.0) if i == 1 else (2.0, -1.0)
        t_i, g, acc = _tc_combine(a, b, partial, ds, t_im2, acc, c_arr[i])
        t_im2, t_im1 = t_im1, t_i

    return acc[:N_NODES]


# speedup vs baseline: 13.7365x; 1.4497x over previous
import functools

import jax
import jax.numpy as jnp
from jax import lax
from jax.experimental import pallas as pl
from jax.experimental.pallas import tpu as pltpu
from jax.experimental.pallas import tpu_sc as plsc

K = 10
N_NODES = 10000
D = 128
NP = 10240
N_EDGES = 320000
NC, NS = 2, 16
W = NC * NS
B = 128
CH = 80
EP = W * CH * B
ROWS_PER_TILE = NP // NS
NB = 4

_mesh = plsc.VectorSubcoreMesh(
    core_axis_name="c", subcore_axis_name="s", num_cores=NC, num_subcores=NS)


@functools.partial(
    pl.kernel,
    out_type=jax.ShapeDtypeStruct((NC, NP), jnp.float32),
    mesh=_mesh,
    scratch_types=[
        pltpu.VMEM((CH, B), jnp.int32),
        pltpu.VMEM((B,), jnp.float32),
        pltpu.VMEM_SHARED((NP,), jnp.float32),
        pltpu.SemaphoreType.DMA,
    ],
)
def _deg_kernel(dst_hbm, ones_hbm, zeros_hbm, out_hbm, dst_v, ones_v, deg_sh,
                ssem):
    c = lax.axis_index("c")
    s = lax.axis_index("s")
    wid = c * NS + s
    pltpu.sync_copy(zeros_hbm.at[pl.ds(s * ROWS_PER_TILE, ROWS_PER_TILE)],
                    deg_sh.at[pl.ds(s * ROWS_PER_TILE, ROWS_PER_TILE)])
    pltpu.sync_copy(dst_hbm.at[wid], dst_v)
    pltpu.sync_copy(ones_hbm, ones_v)
    plsc.subcore_barrier()

    def fire(j, carry):
        pltpu.async_copy(ones_v, deg_sh.at[dst_v.at[j]], ssem, add=True)
        return carry

    def drain(j, carry):
        pltpu.make_async_copy(ones_v, deg_sh.at[dst_v.at[j]], ssem).wait()
        return carry

    lax.fori_loop(0, CH, fire, 0)
    lax.fori_loop(0, CH, drain, 0)
    plsc.subcore_barrier()
    pltpu.sync_copy(deg_sh.at[pl.ds(s * ROWS_PER_TILE, ROWS_PER_TILE)],
                    out_hbm.at[c, pl.ds(s * ROWS_PER_TILE, ROWS_PER_TILE)])


@functools.partial(
    pl.kernel,
    out_type=jax.ShapeDtypeStruct((NC, NP, D), jnp.float32),
    mesh=_mesh,
    scratch_types=[
        pltpu.VMEM((4, B), jnp.int32),
        pltpu.VMEM((4, B), jnp.int32),
        pltpu.VMEM((B, D), jnp.float32),
        pltpu.VMEM((B, D), jnp.float32),
        pltpu.VMEM_SHARED((NP, D), jnp.float32),
        pltpu.SemaphoreType.DMA((4,)),
        pltpu.SemaphoreType.DMA((4,)),
        pltpu.SemaphoreType.DMA((2,)),
    ],
)
def _edge_kernel(g_hbm, src_hbm, dst_hbm, zeros_hbm, out_hbm,
                 sb, db, gb0, gb1, acc_sh, isem, dsem, gsem):
    c = lax.axis_index("c")
    s = lax.axis_index("s")
    wid = c * NS + s
    gbufs = (gb0, gb1)

    def iload_start(jc, slot):
        jc = jnp.minimum(jc, CH - 1)
        pltpu.async_copy(src_hbm.at[wid, jc], sb.at[slot], isem.at[slot])
        pltpu.async_copy(dst_hbm.at[wid, jc], db.at[slot], dsem.at[slot])

    def iload_wait_src(slot):
        pltpu.make_async_copy(
            src_hbm.at[wid, 0], sb.at[slot], isem.at[slot]).wait()

    def iload_wait_dst(slot):
        pltpu.make_async_copy(
            dst_hbm.at[wid, 0], db.at[slot], dsem.at[slot]).wait()

    def gather_start(slot, b):
        pltpu.async_copy(g_hbm.at[sb.at[slot]], gbufs[b], gsem.at[b])

    def gather_wait(slot, b):
        pltpu.make_async_copy(
            g_hbm.at[sb.at[slot]], gbufs[b], gsem.at[b]).wait()

    pltpu.sync_copy(zeros_hbm.at[pl.ds(s * ROWS_PER_TILE, ROWS_PER_TILE)],
                    acc_sh.at[pl.ds(s * ROWS_PER_TILE, ROWS_PER_TILE)])
    for q in range(4):
        iload_start(q, q)
    iload_wait_src(0)
    gather_start(0, 0)
    iload_wait_src(1)
    gather_start(1, 1)
    plsc.subcore_barrier()

    def body(i, carry):
        for u in range(4):
            j = i * 4 + u
            b = u % 2
            gather_wait(u, b)
            iload_wait_dst(u)
            pltpu.sync_copy(gbufs[b], acc_sh.at[db.at[u]], add=True)
            iload_wait_src((u + 2) % 4)
            gather_start((u + 2) % 4, b)
            iload_start(j + 4, u)
        return carry

    lax.fori_loop(0, CH // 4, body, 0)
    for b in range(2):
        gather_wait((CH + b) % 4, b)
    for q in range(2):
        iload_wait_src((CH + 2 + q) % 4)
    for q in range(4):
        iload_wait_dst((CH + q) % 4)
    plsc.subcore_barrier()
    pltpu.sync_copy(acc_sh.at[pl.ds(s * ROWS_PER_TILE, ROWS_PER_TILE)],
                    out_hbm.at[c, pl.ds(s * ROWS_PER_TILE, ROWS_PER_TILE)])


def _setup_body(deg_ref, x_ref, c0_ref, ds_ref, g_ref, acc_ref):
    d = deg_ref[0] + deg_ref[1]
    ds = jnp.where(d > 0.0, lax.rsqrt(jnp.maximum(d, 1e-30)), 0.0)
    ds_ref[...] = ds
    g_ref[...] = ds * x_ref[...]
    acc_ref[...] = c0_ref[0, 0] * x_ref[...]


def _combine_body(a, b, p_ref, ds_ref, tm2_ref, acc_ref, ci_ref,
                  t_ref, g_ref, acco_ref):
    m = p_ref[0] + p_ref[1]
    ds = ds_ref[...]
    t = (a * ds) * m + b * tm2_ref[...]
    acco_ref[...] = acc_ref[...] + ci_ref[0, 0] * t
    t_ref[...] = t
    g_ref[...] = ds * t


_BLK = 256
_GRID = NP // _BLK


def _tc_setup(deg2, x_pad, c0):
    return pl.pallas_call(
        _setup_body,
        grid=(_GRID,),
        in_specs=[
            pl.BlockSpec((NC, _BLK, 1), lambda i: (0, i, 0)),
            pl.BlockSpec((_BLK, D), lambda i: (i, 0)),
            pl.BlockSpec((1, 1), lambda i: (0, 0)),
        ],
        out_specs=[
            pl.BlockSpec((_BLK, 1), lambda i: (i, 0)),
            pl.BlockSpec((_BLK, D), lambda i: (i, 0)),
            pl.BlockSpec((_BLK, D), lambda i: (i, 0)),
        ],
        out_shape=[
            jax.ShapeDtypeStruct((NP, 1), jnp.float32),
            jax.ShapeDtypeStruct((NP, D), jnp.float32),
            jax.ShapeDtypeStruct((NP, D), jnp.float32),
        ],
    )(deg2, x_pad, c0)


def _tc_combine(a, b, partial, ds, t_m2, acc, ci):
    return pl.pallas_call(
        functools.partial(_combine_body, a, b),
        grid=(_GRID,),
        in_specs=[
            pl.BlockSpec((NC, _BLK, D), lambda i: (0, i, 0)),
            pl.BlockSpec((_BLK, 1), lambda i: (i, 0)),
            pl.BlockSpec((_BLK, D), lambda i: (i, 0)),
            pl.BlockSpec((_BLK, D), lambda i: (i, 0)),
            pl.BlockSpec((1, 1), lambda i: (0, 0)),
        ],
        out_specs=[
            pl.BlockSpec((_BLK, D), lambda i: (i, 0)),
            pl.BlockSpec((_BLK, D), lambda i: (i, 0)),
            pl.BlockSpec((_BLK, D), lambda i: (i, 0)),
        ],
        out_shape=[
            jax.ShapeDtypeStruct((NP, D), jnp.float32),
            jax.ShapeDtypeStruct((NP, D), jnp.float32),
            jax.ShapeDtypeStruct((NP, D), jnp.float32),
        ],
    )(partial, ds, t_m2, acc, ci)


def kernel(x, lap_coefs, mf_weights, edge_index):
    src = edge_index[0].astype(jnp.int32)
    dst = edge_index[1].astype(jnp.int32)
    npad = EP - N_EDGES
    pad_src = (jnp.arange(npad, dtype=jnp.int32) * 997) % N_NODES
    pad_dst = N_NODES + (jnp.arange(npad, dtype=jnp.int32) % (NP - N_NODES))
    src3 = jnp.concatenate([src, pad_src]).reshape(W, CH, B)
    dst3 = jnp.concatenate([dst, pad_dst]).reshape(W, CH, B)

    x_pad = jnp.zeros((NP, D), jnp.float32).at[:N_NODES].set(x)
    zeros2 = jnp.zeros((NP, D), jnp.float32)
    zeros1 = jnp.zeros((NP,), jnp.float32)
    ones_b = jnp.ones((B,), jnp.float32)

    lc = jnp.cumprod(jnp.tanh(lap_coefs))
    mw = mf_weights[0, :, 0]
    coefs = mw * jnp.concatenate([jnp.ones((1,), jnp.float32), lc[:-1]])
    c_arr = coefs.reshape(K + 1, 1, 1)

    deg2 = _deg_kernel(dst3, ones_b, zeros1)
    deg2 = deg2.reshape(NC, NP, 1)
    ds, g, acc = _tc_setup(deg2, x_pad, c_arr[0])

    t_im1, t_im2 = x_pad, x_pad
    for i in range(1, K + 1):
        partial = _edge_kernel(g, src3, dst3, zeros2)
        a, b = (1.0, 0.0) if i == 1 else (2.0, -1.0)
        t_i, g, acc = _tc_combine(a, b, partial, ds, t_im2, acc, c_arr[i])
        t_im2, t_im1 = t_im1, t_i

    return acc[:N_NODES]
